# baseline (speedup 1.0000x reference)
import functools

import jax
import jax.numpy as jnp
from jax import lax
from jax.experimental import pallas as pl
from jax.experimental.pallas import tpu as pltpu
from jax.experimental.pallas import tpu_sc as plsc

EMB = 32
NC = 2
NS = 16
NW = NC * NS


def _gather_sc(user_r, item_r, user_table, item_table, B):
    b_per_w = B // NW
    mesh = plsc.VectorSubcoreMesh(core_axis_name="c", subcore_axis_name="s")

    @functools.partial(
        pl.kernel,
        mesh=mesh,
        out_type=(
            jax.ShapeDtypeStruct((B, EMB), jnp.float32),
            jax.ShapeDtypeStruct((B, EMB), jnp.float32),
        ),
        scratch_types=[
            pltpu.SMEM((b_per_w,), jnp.int32),
            pltpu.SMEM((b_per_w,), jnp.int32),
            pltpu.VMEM_SHARED((NS, b_per_w), jnp.int32),
            pltpu.VMEM_SHARED((NS, b_per_w), jnp.int32),
            pltpu.SemaphoreType.DMA,
            pltpu.SemaphoreType.DMA,
        ],
    )
    def gather_kernel(u_idx_hbm, i_idx_hbm, u_tab, i_tab, u_out, i_out,
                      uidx_s, iidx_s, ush_v, ish_v, usem, isem):
        sid = lax.axis_index("s")
        wid = sid * NC + lax.axis_index("c")
        base = wid * b_per_w
        pltpu.sync_copy(u_idx_hbm.at[wid], ush_v.at[sid])
        pltpu.sync_copy(i_idx_hbm.at[wid], ish_v.at[sid])
        pltpu.sync_copy(ush_v.at[sid], uidx_s)
        pltpu.sync_copy(ish_v.at[sid], iidx_s)

        @plsc.parallel_loop(0, b_per_w, unroll=8)
        def _(k):
            pltpu.async_copy(u_tab.at[pl.ds(uidx_s[k], 1)],
                             u_out.at[pl.ds(base + k, 1)], usem)
            pltpu.async_copy(i_tab.at[pl.ds(iidx_s[k], 1)],
                             i_out.at[pl.ds(base + k, 1)], isem)

        pltpu.make_async_copy(u_tab.at[pl.ds(0, b_per_w)],
                              u_out.at[pl.ds(base, b_per_w)], usem).wait()
        pltpu.make_async_copy(i_tab.at[pl.ds(0, b_per_w)],
                              i_out.at[pl.ds(base, b_per_w)], isem).wait()

    return gather_kernel(user_r, item_r, user_table, item_table)


def _mlp_body(u_ref, i_ref, w1u_ref, w1i_ref, b1_ref, w2_ref, b2_ref,
              w3_ref, b3_ref, o_ref):
    x1 = jnp.dot(u_ref[...], w1u_ref[...], preferred_element_type=jnp.float32)
    x1 += jnp.dot(i_ref[...], w1i_ref[...], preferred_element_type=jnp.float32)
    x1 = jnp.maximum(x1 + b1_ref[...], 0.0)
    x2 = jnp.dot(x1, w2_ref[...], preferred_element_type=jnp.float32)
    x2 = jnp.maximum(x2 + b2_ref[...], 0.0)
    logits = jnp.dot(x2, w3_ref[...], preferred_element_type=jnp.float32)
    logits = logits + b3_ref[...]
    m = jnp.max(logits, axis=1, keepdims=True)
    e = jnp.exp(logits - m)
    o_ref[...] = e / jnp.sum(e, axis=1, keepdims=True)


def _mlp_tc(u_emb, i_emb, W1uT, W1iT, b1, W2T, b2, W3T, b3, interpret=False):
    B = u_emb.shape[0]
    BLK = 2048
    n_out = W3T.shape[1]
    full = lambda shape: pl.BlockSpec(shape, lambda i: (0, 0))
    return pl.pallas_call(
        _mlp_body,
        grid=(B // BLK,),
        in_specs=[
            pl.BlockSpec((BLK, EMB), lambda i: (i, 0)),
            pl.BlockSpec((BLK, EMB), lambda i: (i, 0)),
            full(W1uT.shape),
            full(W1iT.shape),
            full(b1.shape),
            full(W2T.shape),
            full(b2.shape),
            full(W3T.shape),
            full(b3.shape),
        ],
        out_specs=pl.BlockSpec((BLK, n_out), lambda i: (i, 0)),
        out_shape=jax.ShapeDtypeStruct((B, n_out), jnp.float32),
        interpret=interpret,
    )(u_emb, i_emb, W1uT, W1iT, b1, W2T, b2, W3T, b3)


def kernel(user, item, user_table, item_table, W1, b1, W2, b2, W3, b3):
    B = user.shape[0]
    b_per_w = B // NW
    user_r = user.astype(jnp.int32).reshape(NW, b_per_w)
    item_r = item.astype(jnp.int32).reshape(NW, b_per_w)
    u_emb, i_emb = _gather_sc(user_r, item_r, user_table, item_table, B)
    W1uT = W1[:, :EMB].T
    W1iT = W1[:, EMB:].T
    return _mlp_tc(u_emb, i_emb, W1uT, W1iT, b1.reshape(1, -1),
                   W2.T, b2.reshape(1, -1), W3.T, b3.reshape(1, -1))


# speedup vs baseline: 1.7944x; 1.7944x over previous
import functools

import jax
import jax.numpy as jnp
from jax import lax
from jax.experimental import pallas as pl
from jax.experimental.pallas import tpu as pltpu
from jax.experimental.pallas import tpu_sc as plsc

EMB = 32
NC = 2
NS = 16
NW = NC * NS
CHUNK = 256


def _gather_sc(user_r, item_r, user_table, item_table, B):
    b_per_w = B // NW
    mesh = plsc.VectorSubcoreMesh(core_axis_name="c", subcore_axis_name="s")

    @functools.partial(
        pl.kernel,
        mesh=mesh,
        out_type=(
            jax.ShapeDtypeStruct((B, EMB), jnp.float32),
            jax.ShapeDtypeStruct((B, EMB), jnp.float32),
        ),
        scratch_types=[
            pltpu.SMEM((b_per_w,), jnp.int32),
            pltpu.SMEM((b_per_w,), jnp.int32),
            pltpu.VMEM_SHARED((NS, b_per_w), jnp.int32),
            pltpu.VMEM_SHARED((NS, b_per_w), jnp.int32),
            pltpu.VMEM((CHUNK, EMB), jnp.float32),
            pltpu.VMEM((CHUNK, EMB), jnp.float32),
            pltpu.SemaphoreType.DMA,
            pltpu.SemaphoreType.DMA,
        ],
    )
    def gather_kernel(u_idx_hbm, i_idx_hbm, u_tab, i_tab, u_out, i_out,
                      uidx_s, iidx_s, ush_v, ish_v, urows_v, irows_v,
                      usem, isem):
        sid = lax.axis_index("s")
        wid = sid * NC + lax.axis_index("c")
        base = wid * b_per_w
        pltpu.sync_copy(u_idx_hbm.at[wid], ush_v.at[sid])
        pltpu.sync_copy(i_idx_hbm.at[wid], ish_v.at[sid])
        pltpu.sync_copy(ush_v.at[sid], uidx_s)
        pltpu.sync_copy(ish_v.at[sid], iidx_s)

        for ch in range(b_per_w // CHUNK):
            off = ch * CHUNK

            @plsc.parallel_loop(0, CHUNK, unroll=8)
            def _(k, off=off):
                pltpu.async_copy(u_tab.at[pl.ds(uidx_s[off + k], 1)],
                                 urows_v.at[pl.ds(k, 1)], usem)
                pltpu.async_copy(i_tab.at[pl.ds(iidx_s[off + k], 1)],
                                 irows_v.at[pl.ds(k, 1)], isem)

            pltpu.make_async_copy(u_tab.at[pl.ds(0, CHUNK)],
                                  urows_v, usem).wait()
            pltpu.make_async_copy(i_tab.at[pl.ds(0, CHUNK)],
                                  irows_v, isem).wait()
            dst = pl.ds(base + off, CHUNK)
            pltpu.sync_copy(urows_v, u_out.at[dst])
            pltpu.sync_copy(irows_v, i_out.at[dst])

    return gather_kernel(user_r, item_r, user_table, item_table)


def _mlp_body(u_ref, i_ref, w1u_ref, w1i_ref, b1_ref, w2_ref, b2_ref,
              w3_ref, b3_ref, o_ref):
    x1 = jnp.dot(u_ref[...], w1u_ref[...], preferred_element_type=jnp.float32)
    x1 += jnp.dot(i_ref[...], w1i_ref[...], preferred_element_type=jnp.float32)
    x1 = jnp.maximum(x1 + b1_ref[...], 0.0)
    x2 = jnp.dot(x1, w2_ref[...], preferred_element_type=jnp.float32)
    x2 = jnp.maximum(x2 + b2_ref[...], 0.0)
    logits = jnp.dot(x2, w3_ref[...], preferred_element_type=jnp.float32)
    logits = logits + b3_ref[...]
    m = jnp.max(logits, axis=1, keepdims=True)
    e = jnp.exp(logits - m)
    o_ref[...] = e / jnp.sum(e, axis=1, keepdims=True)


def _mlp_tc(u_emb, i_emb, W1uT, W1iT, b1, W2T, b2, W3T, b3, interpret=False):
    B = u_emb.shape[0]
    BLK = 2048
    n_out = W3T.shape[1]
    full = lambda shape: pl.BlockSpec(shape, lambda i: (0, 0))
    return pl.pallas_call(
        _mlp_body,
        grid=(B // BLK,),
        in_specs=[
            pl.BlockSpec((BLK, EMB), lambda i: (i, 0)),
            pl.BlockSpec((BLK, EMB), lambda i: (i, 0)),
            full(W1uT.shape),
            full(W1iT.shape),
            full(b1.shape),
            full(W2T.shape),
            full(b2.shape),
            full(W3T.shape),
            full(b3.shape),
        ],
        out_specs=pl.BlockSpec((BLK, n_out), lambda i: (i, 0)),
        out_shape=jax.ShapeDtypeStruct((B, n_out), jnp.float32),
        interpret=interpret,
    )(u_emb, i_emb, W1uT, W1iT, b1, W2T, b2, W3T, b3)


def kernel(user, item, user_table, item_table, W1, b1, W2, b2, W3, b3):
    B = user.shape[0]
    b_per_w = B // NW
    user_r = user.astype(jnp.int32).reshape(NW, b_per_w)
    item_r = item.astype(jnp.int32).reshape(NW, b_per_w)
    u_emb, i_emb = _gather_sc(user_r, item_r, user_table, item_table, B)
    W1uT = W1[:, :EMB].T
    W1iT = W1[:, EMB:].T
    return _mlp_tc(u_emb, i_emb, W1uT, W1iT, b1.reshape(1, -1),
                   W2.T, b2.reshape(1, -1), W3.T, b3.reshape(1, -1))
